# baseline (speedup 1.0000x reference)
import functools

import jax
import jax.numpy as jnp
from jax import lax
from jax.experimental import pallas as pl
from jax.experimental.pallas import tpu as pltpu
from jax.experimental.pallas import tpu_sc as plsc

N = 10000
E = 320000
D = 128
H = 128
K = 2
C = 40

NP = 10240
CH = 128
NCHUNK = 2560
EP = NCHUNK * CH
CP = 128

_MESH = plsc.VectorSubcoreMesh(
    core_axis_name="c", subcore_axis_name="s", num_cores=2, num_subcores=16
)


def _count_body(dst_flat, cnt_out, idx_v, cnt_v):
    c = lax.axis_index("c")
    s = lax.axis_index("s")
    wid = s * 2 + c
    pltpu.sync_copy(dst_flat.at[pl.ds(wid * (EP // 32), EP // 32)], idx_v)

    def zero(i, carry):
        cnt_v[pl.ds(i * 16, 16)] = jnp.zeros((16,), jnp.float32)
        return carry

    lax.fori_loop(0, NP // 16, zero, 0)

    ones = jnp.ones((16,), jnp.float32)

    def body(i, carry):
        idx = idx_v[pl.ds(i * 16, 16)]
        plsc.addupdate_scatter(cnt_v, [idx], ones)
        return carry

    lax.fori_loop(0, EP // 32 // 16, body, 0)
    pltpu.sync_copy(cnt_v, cnt_out.at[wid])


def _count_kernel(dst_flat):
    return pl.kernel(
        _count_body,
        out_type=jax.ShapeDtypeStruct((32, NP), jnp.float32),
        mesh=_MESH,
        scratch_types=[
            pltpu.VMEM((EP // 32,), jnp.int32),
            pltpu.VMEM((NP,), jnp.float32),
        ],
        compiler_params=pltpu.CompilerParams(needs_layout_passes=False),
    )(dst_flat)


def _pipelined_block(table, acc_sp, src_v, dst_v, buf0, buf1, sem0, sem1, nch):
    pltpu.async_copy(table.at[src_v.at[0]], buf0, sem0)

    def inner(jj, carry):
        j0 = 2 * jj
        pltpu.async_copy(table.at[src_v.at[j0 + 1]], buf1, sem1)
        pltpu.make_async_copy(table.at[src_v.at[j0]], buf0, sem0).wait()
        pltpu.sync_copy(buf0, acc_sp.at[dst_v.at[j0]], add=True)

        @pl.when(jj + 1 < nch // 2)
        def _():
            pltpu.async_copy(table.at[src_v.at[j0 + 2]], buf0, sem0)

        pltpu.make_async_copy(table.at[src_v.at[j0 + 1]], buf1, sem1).wait()
        pltpu.sync_copy(buf1, acc_sp.at[dst_v.at[j0 + 1]], add=True)
        return carry

    lax.fori_loop(0, nch // 2, inner, 0)


def _prop128_body(g0, g1, src2d, dst2d, zeros_hbm, acc_out,
                  src_v, dst_v, buf0, buf1, sem0, sem1, acc_sp):
    c = lax.axis_index("c")
    s = lax.axis_index("s")
    per_tile = NCHUNK // 16
    rows = NP // 16
    pltpu.sync_copy(zeros_hbm.at[pl.ds(s * rows, rows)],
                    acc_sp.at[pl.ds(s * rows, rows)])
    plsc.subcore_barrier()

    def run(table):
        def outer(ob, carry):
            base = s * per_tile + ob * _SB
            pltpu.sync_copy(src2d.at[pl.ds(base, _SB)], src_v)
            pltpu.sync_copy(dst2d.at[pl.ds(base, _SB)], dst_v)
            _pipelined_block(table, acc_sp, src_v, dst_v,
                             buf0, buf1, sem0, sem1, _SB)
            return carry

        lax.fori_loop(0, per_tile // _SB, outer, 0)

    @pl.when(c == 0)
    def _():
        run(g0)

    @pl.when(c == 1)
    def _():
        run(g1)

    plsc.subcore_barrier()
    pltpu.sync_copy(acc_sp.at[pl.ds(s * rows, rows)],
                    acc_out.at[pl.ds(c * NP + s * rows, rows)])


_SB = 32


def _prop128_kernel(g0, g1, src2d, dst2d, zeros_hbm):
    return pl.kernel(
        _prop128_body,
        out_type=jax.ShapeDtypeStruct((2 * NP, H), jnp.float32),
        mesh=_MESH,
        scratch_types=[
            pltpu.VMEM((_SB, CH), jnp.int32),
            pltpu.VMEM((_SB, CH), jnp.int32),
            pltpu.VMEM((CH, H), jnp.float32),
            pltpu.VMEM((CH, H), jnp.float32),
            pltpu.SemaphoreType.DMA,
            pltpu.SemaphoreType.DMA,
            pltpu.VMEM_SHARED((NP, H), jnp.float32),
        ],
    )(g0, g1, src2d, dst2d, zeros_hbm)


def _prop64_body(u_hbm, src2d, dst2d, zeros_hbm, acc_out,
                 src_v, dst_v, buf0, buf1, sem0, sem1, acc_sp):
    c = lax.axis_index("c")
    s = lax.axis_index("s")
    per_tile = NCHUNK // 32
    tile_base = c * (NCHUNK // 2) + s * per_tile
    rows = NP // 16
    pltpu.sync_copy(zeros_hbm.at[pl.ds(s * rows, rows)],
                    acc_sp.at[pl.ds(s * rows, rows)])
    plsc.subcore_barrier()

    def outer(ob, carry):
        base = tile_base + ob * _SB2
        pltpu.sync_copy(src2d.at[pl.ds(base, _SB2)], src_v)
        pltpu.sync_copy(dst2d.at[pl.ds(base, _SB2)], dst_v)
        _pipelined_block(u_hbm, acc_sp, src_v, dst_v,
                         buf0, buf1, sem0, sem1, _SB2)
        return carry

    lax.fori_loop(0, per_tile // _SB2, outer, 0)

    plsc.subcore_barrier()
    pltpu.sync_copy(acc_sp.at[pl.ds(s * rows, rows)],
                    acc_out.at[pl.ds(c * NP + s * rows, rows)])


_SB2 = 16


def _prop64_kernel(u, src2d, dst2d, zeros64):
    return pl.kernel(
        _prop64_body,
        out_type=jax.ShapeDtypeStruct((2 * NP, CP), jnp.float32),
        mesh=_MESH,
        scratch_types=[
            pltpu.VMEM((_SB2, CH), jnp.int32),
            pltpu.VMEM((_SB2, CH), jnp.int32),
            pltpu.VMEM((CH, CP), jnp.float32),
            pltpu.VMEM((CH, CP), jnp.float32),
            pltpu.SemaphoreType.DMA,
            pltpu.SemaphoreType.DMA,
            pltpu.VMEM_SHARED((NP, CP), jnp.float32),
        ],
    )(u, src2d, dst2d, zeros64)


_BR = 512


def _tc1_body(cnt_ref, x_ref, w_ref, g0_ref, g1_ref, dinv_ref):
    cnt = jnp.sum(cnt_ref[...], axis=0)
    dinv = lax.rsqrt(cnt + 1.0)
    dcol = dinv[:, None]
    h0 = jnp.dot(x_ref[0], w_ref[0], preferred_element_type=jnp.float32)
    h1 = jnp.dot(x_ref[1], w_ref[1], preferred_element_type=jnp.float32)
    g0_ref[...] = h0 * dcol
    g1_ref[...] = h1 * dcol
    dinv_ref[...] = dcol


def _tc1(cnt32, xp, W_hidden):
    return pl.pallas_call(
        _tc1_body,
        grid=(NP // _BR,),
        in_specs=[
            pl.BlockSpec((32, _BR), lambda i: (0, i)),
            pl.BlockSpec((K, _BR, D), lambda i: (0, i, 0)),
            pl.BlockSpec((K, D, H), lambda i: (0, 0, 0)),
        ],
        out_specs=[
            pl.BlockSpec((_BR, H), lambda i: (i, 0)),
            pl.BlockSpec((_BR, H), lambda i: (i, 0)),
            pl.BlockSpec((_BR, 1), lambda i: (i, 0)),
        ],
        out_shape=[
            jax.ShapeDtypeStruct((NP, H), jnp.float32),
            jax.ShapeDtypeStruct((NP, H), jnp.float32),
            jax.ShapeDtypeStruct((NP, 1), jnp.float32),
        ],
    )(cnt32, xp, W_hidden)


def _tc2_body(a0_ref, a1_ref, g0_ref, g1_ref, dinv_ref, b_ref, wo_ref, u_ref):
    dinv = dinv_ref[...]
    h0 = jnp.maximum((a0_ref[...] + g0_ref[...]) * dinv + b_ref[0], 0.0)
    h1 = jnp.maximum((a1_ref[...] + g1_ref[...]) * dinv + b_ref[1], 0.0)
    wo = wo_ref[...]
    z = jnp.dot(h0, wo[:H], preferred_element_type=jnp.float32)
    z = z + jnp.dot(h1, wo[H:], preferred_element_type=jnp.float32)
    u_ref[...] = z * dinv


def _tc2(acc, g0, g1, dinv, b_hidden, wo_pad):
    nb = NP // _BR
    return pl.pallas_call(
        _tc2_body,
        grid=(nb,),
        in_specs=[
            pl.BlockSpec((_BR, H), lambda i: (i, 0)),
            pl.BlockSpec((_BR, H), lambda i: (i + nb, 0)),
            pl.BlockSpec((_BR, H), lambda i: (i, 0)),
            pl.BlockSpec((_BR, H), lambda i: (i, 0)),
            pl.BlockSpec((_BR, 1), lambda i: (i, 0)),
            pl.BlockSpec((K, H), lambda i: (0, 0)),
            pl.BlockSpec((K * H, CP), lambda i: (0, 0)),
        ],
        out_specs=pl.BlockSpec((_BR, CP), lambda i: (i, 0)),
        out_shape=jax.ShapeDtypeStruct((NP, CP), jnp.float32),
    )(acc, acc, g0, g1, dinv, b_hidden, wo_pad)


def _tc3_body(a0_ref, a1_ref, u_ref, dinv_ref, b_ref, out_ref):
    t = (a0_ref[...] + a1_ref[...] + u_ref[...]) * dinv_ref[...]
    out_ref[...] = t[:, :C] + b_ref[...]


def _tc3(acc2, u, dinv, b_out):
    nb = NP // _BR
    return pl.pallas_call(
        _tc3_body,
        grid=(nb,),
        in_specs=[
            pl.BlockSpec((_BR, CP), lambda i: (i, 0)),
            pl.BlockSpec((_BR, CP), lambda i: (i + nb, 0)),
            pl.BlockSpec((_BR, CP), lambda i: (i, 0)),
            pl.BlockSpec((_BR, 1), lambda i: (i, 0)),
            pl.BlockSpec((1, C), lambda i: (0, 0)),
        ],
        out_specs=pl.BlockSpec((_BR, C), lambda i: (i, 0)),
        out_shape=jax.ShapeDtypeStruct((NP, C), jnp.float32),
    )(acc2, acc2, u, dinv, b_out)


def kernel(x_list, adj, W_hidden, b_hidden, W_out, b_out):
    src = jnp.concatenate([adj[0], jnp.zeros((EP - E,), jnp.int32)])
    dst = jnp.concatenate([adj[1], jnp.full((EP - E,), NP - 1, jnp.int32)])
    src, dst = jax.lax.sort_key_val(src, dst)
    src2d = src.reshape(NCHUNK, CH)
    dst2d = dst.reshape(NCHUNK, CH)
    xp = jnp.zeros((K, NP, D), jnp.float32).at[:, :N].set(x_list)
    wo_pad = jnp.zeros((K * H, CP), jnp.float32).at[:, :C].set(W_out)
    zeros128 = jnp.zeros((NP, H), jnp.float32)

    cnt32 = _count_kernel(dst)
    g0, g1, dinv = _tc1(cnt32, xp, W_hidden)

    acc = _prop128_kernel(g0, g1, src2d, dst2d, zeros128)
    u = _tc2(acc, g0, g1, dinv, b_hidden, wo_pad)

    acc2 = _prop64_kernel(u, src2d, dst2d, zeros128)
    out = _tc3(acc2, u, dinv, b_out.reshape(1, C))
    return out[:N]


# speedup vs baseline: 2.1485x; 2.1485x over previous
import functools

import jax
import jax.numpy as jnp
from jax import lax
from jax.experimental import pallas as pl
from jax.experimental.pallas import tpu as pltpu
from jax.experimental.pallas import tpu_sc as plsc

N = 10000
E = 320000
D = 128
H = 128
K = 2
C = 40

NP = 10240
CH = 128
NCHUNK = 2560
EP = NCHUNK * CH
CP = 64

_MESH = plsc.VectorSubcoreMesh(
    core_axis_name="c", subcore_axis_name="s", num_cores=2, num_subcores=16
)


def _count_body(dst_flat, cnt_out, idx_v, cnt_v):
    c = lax.axis_index("c")
    s = lax.axis_index("s")
    wid = s * 2 + c
    pltpu.sync_copy(dst_flat.at[pl.ds(wid * (EP // 32), EP // 32)], idx_v)

    def zero(i, carry):
        cnt_v[pl.ds(i * 16, 16)] = jnp.zeros((16,), jnp.float32)
        return carry

    lax.fori_loop(0, NP // 16, zero, 0)

    ones = jnp.ones((16,), jnp.float32)

    def body(i, carry):
        idx = idx_v[pl.ds(i * 16, 16)]
        plsc.addupdate_scatter(cnt_v, [idx], ones)
        return carry

    lax.fori_loop(0, EP // 32 // 16, body, 0)
    pltpu.sync_copy(cnt_v, cnt_out.at[wid])


def _count_kernel(dst_flat):
    return pl.kernel(
        _count_body,
        out_type=jax.ShapeDtypeStruct((32, NP), jnp.float32),
        mesh=_MESH,
        scratch_types=[
            pltpu.VMEM((EP // 32,), jnp.int32),
            pltpu.VMEM((NP,), jnp.float32),
        ],
        compiler_params=pltpu.CompilerParams(needs_layout_passes=False),
    )(dst_flat)


def _pipelined_block(table, acc_sp, src_v, dst_v, buf0, buf1, sem0, sem1, nch):
    pltpu.async_copy(table.at[src_v.at[0]], buf0, sem0)

    def inner(jj, carry):
        j0 = 2 * jj
        pltpu.async_copy(table.at[src_v.at[j0 + 1]], buf1, sem1)
        pltpu.make_async_copy(table.at[src_v.at[j0]], buf0, sem0).wait()
        pltpu.sync_copy(buf0, acc_sp.at[dst_v.at[j0]], add=True)

        @pl.when(jj + 1 < nch // 2)
        def _():
            pltpu.async_copy(table.at[src_v.at[j0 + 2]], buf0, sem0)

        pltpu.make_async_copy(table.at[src_v.at[j0 + 1]], buf1, sem1).wait()
        pltpu.sync_copy(buf1, acc_sp.at[dst_v.at[j0 + 1]], add=True)
        return carry

    lax.fori_loop(0, nch // 2, inner, 0)


def _prop128_body(g0, g1, src2d, dst2d, zeros_hbm, acc_out,
                  src_v, dst_v, buf0, buf1, sem0, sem1, acc_sp):
    c = lax.axis_index("c")
    s = lax.axis_index("s")
    per_tile = NCHUNK // 16
    rows = NP // 16
    pltpu.sync_copy(zeros_hbm.at[pl.ds(s * rows, rows)],
                    acc_sp.at[pl.ds(s * rows, rows)])
    plsc.subcore_barrier()

    def run(table):
        def outer(ob, carry):
            base = s * per_tile + ob * _SB
            pltpu.sync_copy(src2d.at[pl.ds(base, _SB)], src_v)
            pltpu.sync_copy(dst2d.at[pl.ds(base, _SB)], dst_v)
            _pipelined_block(table, acc_sp, src_v, dst_v,
                             buf0, buf1, sem0, sem1, _SB)
            return carry

        lax.fori_loop(0, per_tile // _SB, outer, 0)

    @pl.when(c == 0)
    def _():
        run(g0)

    @pl.when(c == 1)
    def _():
        run(g1)

    plsc.subcore_barrier()
    pltpu.sync_copy(acc_sp.at[pl.ds(s * rows, rows)],
                    acc_out.at[pl.ds(c * NP + s * rows, rows)])


_SB = 32


def _prop128_kernel(g0, g1, src2d, dst2d, zeros_hbm):
    return pl.kernel(
        _prop128_body,
        out_type=jax.ShapeDtypeStruct((2 * NP, H), jnp.float32),
        mesh=_MESH,
        scratch_types=[
            pltpu.VMEM((_SB, CH), jnp.int32),
            pltpu.VMEM((_SB, CH), jnp.int32),
            pltpu.VMEM((CH, H), jnp.float32),
            pltpu.VMEM((CH, H), jnp.float32),
            pltpu.SemaphoreType.DMA,
            pltpu.SemaphoreType.DMA,
            pltpu.VMEM_SHARED((NP, H), jnp.float32),
        ],
    )(g0, g1, src2d, dst2d, zeros_hbm)


def _prop64_body(u_hbm, src2d, dst2d, zeros_hbm, acc_out,
                 src_v, dst_v, buf0, buf1, sem0, sem1, acc_sp):
    c = lax.axis_index("c")
    s = lax.axis_index("s")
    per_tile = NCHUNK // 32
    tile_base = c * (NCHUNK // 2) + s * per_tile
    rows = NP // 16
    pltpu.sync_copy(zeros_hbm.at[pl.ds(s * rows, rows)],
                    acc_sp.at[pl.ds(s * rows, rows)])
    plsc.subcore_barrier()

    def outer(ob, carry):
        base = tile_base + ob * _SB2
        pltpu.sync_copy(src2d.at[pl.ds(base, _SB2)], src_v)
        pltpu.sync_copy(dst2d.at[pl.ds(base, _SB2)], dst_v)
        _pipelined_block(u_hbm, acc_sp, src_v, dst_v,
                         buf0, buf1, sem0, sem1, _SB2)
        return carry

    lax.fori_loop(0, per_tile // _SB2, outer, 0)

    plsc.subcore_barrier()
    pltpu.sync_copy(acc_sp.at[pl.ds(s * rows, rows)],
                    acc_out.at[pl.ds(c * NP + s * rows, rows)])


_SB2 = 16


def _prop64_kernel(u, src2d, dst2d, zeros64):
    return pl.kernel(
        _prop64_body,
        out_type=jax.ShapeDtypeStruct((2 * NP, CP), jnp.float32),
        mesh=_MESH,
        scratch_types=[
            pltpu.VMEM((_SB2, CH), jnp.int32),
            pltpu.VMEM((_SB2, CH), jnp.int32),
            pltpu.VMEM((CH, CP), jnp.float32),
            pltpu.VMEM((CH, CP), jnp.float32),
            pltpu.SemaphoreType.DMA,
            pltpu.SemaphoreType.DMA,
            pltpu.VMEM_SHARED((NP, CP), jnp.float32),
        ],
        compiler_params=pltpu.CompilerParams(use_tc_tiling_on_sc=False),
    )(u, src2d, dst2d, zeros64)


_BR = 512


def _tc1_body(cnt_ref, x_ref, w_ref, g0_ref, g1_ref, dinv_ref):
    cnt = jnp.sum(cnt_ref[...], axis=0)
    dinv = lax.rsqrt(cnt + 1.0)
    dcol = dinv[:, None]
    h0 = jnp.dot(x_ref[0], w_ref[0], preferred_element_type=jnp.float32)
    h1 = jnp.dot(x_ref[1], w_ref[1], preferred_element_type=jnp.float32)
    g0_ref[...] = h0 * dcol
    g1_ref[...] = h1 * dcol
    dinv_ref[...] = dcol


def _tc1(cnt32, xp, W_hidden):
    return pl.pallas_call(
        _tc1_body,
        grid=(NP // _BR,),
        in_specs=[
            pl.BlockSpec((32, _BR), lambda i: (0, i)),
            pl.BlockSpec((K, _BR, D), lambda i: (0, i, 0)),
            pl.BlockSpec((K, D, H), lambda i: (0, 0, 0)),
        ],
        out_specs=[
            pl.BlockSpec((_BR, H), lambda i: (i, 0)),
            pl.BlockSpec((_BR, H), lambda i: (i, 0)),
            pl.BlockSpec((_BR, 1), lambda i: (i, 0)),
        ],
        out_shape=[
            jax.ShapeDtypeStruct((NP, H), jnp.float32),
            jax.ShapeDtypeStruct((NP, H), jnp.float32),
            jax.ShapeDtypeStruct((NP, 1), jnp.float32),
        ],
    )(cnt32, xp, W_hidden)


def _tc2_body(a0_ref, a1_ref, g0_ref, g1_ref, dinv_ref, b_ref, wo_ref, u_ref):
    dinv = dinv_ref[...]
    h0 = jnp.maximum((a0_ref[...] + g0_ref[...]) * dinv + b_ref[0], 0.0)
    h1 = jnp.maximum((a1_ref[...] + g1_ref[...]) * dinv + b_ref[1], 0.0)
    wo = wo_ref[...]
    z = jnp.dot(h0, wo[:H], preferred_element_type=jnp.float32)
    z = z + jnp.dot(h1, wo[H:], preferred_element_type=jnp.float32)
    u_ref[...] = z * dinv


def _tc2(acc, g0, g1, dinv, b_hidden, wo_pad):
    nb = NP // _BR
    return pl.pallas_call(
        _tc2_body,
        grid=(nb,),
        in_specs=[
            pl.BlockSpec((_BR, H), lambda i: (i, 0)),
            pl.BlockSpec((_BR, H), lambda i: (i + nb, 0)),
            pl.BlockSpec((_BR, H), lambda i: (i, 0)),
            pl.BlockSpec((_BR, H), lambda i: (i, 0)),
            pl.BlockSpec((_BR, 1), lambda i: (i, 0)),
            pl.BlockSpec((K, H), lambda i: (0, 0)),
            pl.BlockSpec((K * H, CP), lambda i: (0, 0)),
        ],
        out_specs=pl.BlockSpec((_BR, CP), lambda i: (i, 0)),
        out_shape=jax.ShapeDtypeStruct((NP, CP), jnp.float32),
    )(acc, acc, g0, g1, dinv, b_hidden, wo_pad)


def _tc3_body(a0_ref, a1_ref, u_ref, dinv_ref, b_ref, out_ref):
    t = (a0_ref[...] + a1_ref[...] + u_ref[...]) * dinv_ref[...]
    out_ref[...] = t[:, :C] + b_ref[...]


def _tc3(acc2, u, dinv, b_out):
    nb = NP // _BR
    return pl.pallas_call(
        _tc3_body,
        grid=(nb,),
        in_specs=[
            pl.BlockSpec((_BR, CP), lambda i: (i, 0)),
            pl.BlockSpec((_BR, CP), lambda i: (i + nb, 0)),
            pl.BlockSpec((_BR, CP), lambda i: (i, 0)),
            pl.BlockSpec((_BR, 1), lambda i: (i, 0)),
            pl.BlockSpec((1, C), lambda i: (0, 0)),
        ],
        out_specs=pl.BlockSpec((_BR, C), lambda i: (i, 0)),
        out_shape=jax.ShapeDtypeStruct((NP, C), jnp.float32),
    )(acc2, acc2, u, dinv, b_out)


def kernel(x_list, adj, W_hidden, b_hidden, W_out, b_out):
    src = jnp.concatenate([adj[0], jnp.zeros((EP - E,), jnp.int32)])
    dst = jnp.concatenate([adj[1], jnp.full((EP - E,), NP - 1, jnp.int32)])
    src2d = src.reshape(NCHUNK, CH)
    dst2d = dst.reshape(NCHUNK, CH)
    xp = jnp.zeros((K, NP, D), jnp.float32).at[:, :N].set(x_list)
    wo_pad = jnp.zeros((K * H, CP), jnp.float32).at[:, :C].set(W_out)
    zeros128 = jnp.zeros((NP, H), jnp.float32)
    zeros64 = jnp.zeros((NP, CP), jnp.float32)

    cnt32 = _count_kernel(dst)
    g0, g1, dinv = _tc1(cnt32, xp, W_hidden)

    acc = _prop128_kernel(g0, g1, src2d, dst2d, zeros128)
    u = _tc2(acc, g0, g1, dinv, b_hidden, wo_pad)

    acc2 = _prop64_kernel(u, src2d, dst2d, zeros64)
    out = _tc3(acc2, u, dinv, b_out.reshape(1, C))
    return out[:N]


# speedup vs baseline: 2.2713x; 1.0572x over previous
import functools

import jax
import jax.numpy as jnp
from jax import lax
from jax.experimental import pallas as pl
from jax.experimental.pallas import tpu as pltpu
from jax.experimental.pallas import tpu_sc as plsc

N = 10000
E = 320000
D = 128
H = 128
K = 2
C = 40

NP = 10240
CH = 128
NCHUNK = 2560
EP = NCHUNK * CH
CP = 48

_MESH = plsc.VectorSubcoreMesh(
    core_axis_name="c", subcore_axis_name="s", num_cores=2, num_subcores=16
)


def _count_body(dst_flat, cnt_out, idx_v, cnt_v):
    c = lax.axis_index("c")
    s = lax.axis_index("s")
    wid = s * 2 + c
    pltpu.sync_copy(dst_flat.at[pl.ds(wid * (EP // 32), EP // 32)], idx_v)

    def zero(i, carry):
        cnt_v[pl.ds(i * 16, 16)] = jnp.zeros((16,), jnp.float32)
        return carry

    lax.fori_loop(0, NP // 16, zero, 0)

    ones = jnp.ones((16,), jnp.float32)

    def body(i, carry):
        idx = idx_v[pl.ds(i * 16, 16)]
        plsc.addupdate_scatter(cnt_v, [idx], ones)
        return carry

    lax.fori_loop(0, EP // 32 // 16, body, 0)
    pltpu.sync_copy(cnt_v, cnt_out.at[wid])


def _count_kernel(dst_flat):
    return pl.kernel(
        _count_body,
        out_type=jax.ShapeDtypeStruct((32, NP), jnp.float32),
        mesh=_MESH,
        scratch_types=[
            pltpu.VMEM((EP // 32,), jnp.int32),
            pltpu.VMEM((NP,), jnp.float32),
        ],
        compiler_params=pltpu.CompilerParams(needs_layout_passes=False),
    )(dst_flat)


def _pipelined_block(table, acc_sp, src_v, dst_v, buf0, buf1, sem0, sem1, nch):
    pltpu.async_copy(table.at[src_v.at[0]], buf0, sem0)

    def inner(jj, carry):
        j0 = 2 * jj
        pltpu.async_copy(table.at[src_v.at[j0 + 1]], buf1, sem1)
        pltpu.make_async_copy(table.at[src_v.at[j0]], buf0, sem0).wait()
        pltpu.sync_copy(buf0, acc_sp.at[dst_v.at[j0]], add=True)

        @pl.when(jj + 1 < nch // 2)
        def _():
            pltpu.async_copy(table.at[src_v.at[j0 + 2]], buf0, sem0)

        pltpu.make_async_copy(table.at[src_v.at[j0 + 1]], buf1, sem1).wait()
        pltpu.sync_copy(buf1, acc_sp.at[dst_v.at[j0 + 1]], add=True)
        return carry

    lax.fori_loop(0, nch // 2, inner, 0)


def _prop128_body(g0, g1, src2d, dst2d, zeros_hbm, acc_out,
                  src_v, dst_v, buf0, buf1, sem0, sem1, acc_sp):
    c = lax.axis_index("c")
    s = lax.axis_index("s")
    per_tile = NCHUNK // 16
    rows = NP // 16
    pltpu.sync_copy(zeros_hbm.at[pl.ds(s * rows, rows)],
                    acc_sp.at[pl.ds(s * rows, rows)])
    plsc.subcore_barrier()

    def run(table):
        def outer(ob, carry):
            base = s * per_tile + ob * _SB
            pltpu.sync_copy(src2d.at[pl.ds(base, _SB)], src_v)
            pltpu.sync_copy(dst2d.at[pl.ds(base, _SB)], dst_v)
            _pipelined_block(table, acc_sp, src_v, dst_v,
                             buf0, buf1, sem0, sem1, _SB)
            return carry

        lax.fori_loop(0, per_tile // _SB, outer, 0)

    @pl.when(c == 0)
    def _():
        run(g0)

    @pl.when(c == 1)
    def _():
        run(g1)

    plsc.subcore_barrier()
    pltpu.sync_copy(acc_sp.at[pl.ds(s * rows, rows)],
                    acc_out.at[pl.ds(c * NP + s * rows, rows)])


_SB = 32


def _prop128_kernel(g0, g1, src2d, dst2d, zeros_hbm):
    return pl.kernel(
        _prop128_body,
        out_type=jax.ShapeDtypeStruct((2 * NP, H), jnp.float32),
        mesh=_MESH,
        scratch_types=[
            pltpu.VMEM((_SB, CH), jnp.int32),
            pltpu.VMEM((_SB, CH), jnp.int32),
            pltpu.VMEM((CH, H), jnp.float32),
            pltpu.VMEM((CH, H), jnp.float32),
            pltpu.SemaphoreType.DMA,
            pltpu.SemaphoreType.DMA,
            pltpu.VMEM_SHARED((NP, H), jnp.float32),
        ],
    )(g0, g1, src2d, dst2d, zeros_hbm)


def _prop64_body(u_hbm, src2d, dst2d, zeros_hbm, acc_out,
                 src_v, dst_v, buf0, buf1, sem0, sem1, acc_sp):
    c = lax.axis_index("c")
    s = lax.axis_index("s")
    per_tile = NCHUNK // 32
    tile_base = c * (NCHUNK // 2) + s * per_tile
    rows = NP // 16
    pltpu.sync_copy(zeros_hbm.at[pl.ds(s * rows, rows)],
                    acc_sp.at[pl.ds(s * rows, rows)])
    plsc.subcore_barrier()

    def outer(ob, carry):
        base = tile_base + ob * _SB2
        pltpu.sync_copy(src2d.at[pl.ds(base, _SB2)], src_v)
        pltpu.sync_copy(dst2d.at[pl.ds(base, _SB2)], dst_v)
        _pipelined_block(u_hbm, acc_sp, src_v, dst_v,
                         buf0, buf1, sem0, sem1, _SB2)
        return carry

    lax.fori_loop(0, per_tile // _SB2, outer, 0)

    plsc.subcore_barrier()
    pltpu.sync_copy(acc_sp.at[pl.ds(s * rows, rows)],
                    acc_out.at[pl.ds(c * NP + s * rows, rows)])


_SB2 = 16


def _prop64_kernel(u, src2d, dst2d, zeros64):
    return pl.kernel(
        _prop64_body,
        out_type=jax.ShapeDtypeStruct((2 * NP, CP), jnp.float32),
        mesh=_MESH,
        scratch_types=[
            pltpu.VMEM((_SB2, CH), jnp.int32),
            pltpu.VMEM((_SB2, CH), jnp.int32),
            pltpu.VMEM((CH, CP), jnp.float32),
            pltpu.VMEM((CH, CP), jnp.float32),
            pltpu.SemaphoreType.DMA,
            pltpu.SemaphoreType.DMA,
            pltpu.VMEM_SHARED((NP, CP), jnp.float32),
        ],
        compiler_params=pltpu.CompilerParams(use_tc_tiling_on_sc=False),
    )(u, src2d, dst2d, zeros64)


_BR = 512


def _tc1_body(cnt_ref, x_ref, w_ref, g0_ref, g1_ref, dinv_ref):
    cnt = jnp.sum(cnt_ref[...], axis=0)
    dinv = lax.rsqrt(cnt + 1.0)
    dcol = dinv[:, None]
    h0 = jnp.dot(x_ref[0], w_ref[0], preferred_element_type=jnp.float32)
    h1 = jnp.dot(x_ref[1], w_ref[1], preferred_element_type=jnp.float32)
    g0_ref[...] = h0 * dcol
    g1_ref[...] = h1 * dcol
    dinv_ref[...] = dcol


def _tc1(cnt32, xp, W_hidden):
    return pl.pallas_call(
        _tc1_body,
        grid=(NP // _BR,),
        in_specs=[
            pl.BlockSpec((32, _BR), lambda i: (0, i)),
            pl.BlockSpec((K, _BR, D), lambda i: (0, i, 0)),
            pl.BlockSpec((K, D, H), lambda i: (0, 0, 0)),
        ],
        out_specs=[
            pl.BlockSpec((_BR, H), lambda i: (i, 0)),
            pl.BlockSpec((_BR, H), lambda i: (i, 0)),
            pl.BlockSpec((_BR, 1), lambda i: (i, 0)),
        ],
        out_shape=[
            jax.ShapeDtypeStruct((NP, H), jnp.float32),
            jax.ShapeDtypeStruct((NP, H), jnp.float32),
            jax.ShapeDtypeStruct((NP, 1), jnp.float32),
        ],
    )(cnt32, xp, W_hidden)


def _tc2_body(a0_ref, a1_ref, g0_ref, g1_ref, dinv_ref, b_ref, wo_ref, u_ref):
    dinv = dinv_ref[...]
    h0 = jnp.maximum((a0_ref[...] + g0_ref[...]) * dinv + b_ref[0], 0.0)
    h1 = jnp.maximum((a1_ref[...] + g1_ref[...]) * dinv + b_ref[1], 0.0)
    wo = wo_ref[...]
    z = jnp.dot(h0, wo[:H], preferred_element_type=jnp.float32)
    z = z + jnp.dot(h1, wo[H:], preferred_element_type=jnp.float32)
    u_ref[...] = z * dinv


def _tc2(acc, g0, g1, dinv, b_hidden, wo_pad):
    nb = NP // _BR
    return pl.pallas_call(
        _tc2_body,
        grid=(nb,),
        in_specs=[
            pl.BlockSpec((_BR, H), lambda i: (i, 0)),
            pl.BlockSpec((_BR, H), lambda i: (i + nb, 0)),
            pl.BlockSpec((_BR, H), lambda i: (i, 0)),
            pl.BlockSpec((_BR, H), lambda i: (i, 0)),
            pl.BlockSpec((_BR, 1), lambda i: (i, 0)),
            pl.BlockSpec((K, H), lambda i: (0, 0)),
            pl.BlockSpec((K * H, CP), lambda i: (0, 0)),
        ],
        out_specs=pl.BlockSpec((_BR, CP), lambda i: (i, 0)),
        out_shape=jax.ShapeDtypeStruct((NP, CP), jnp.float32),
    )(acc, acc, g0, g1, dinv, b_hidden, wo_pad)


def _tc3_body(a0_ref, a1_ref, u_ref, dinv_ref, b_ref, out_ref):
    t = (a0_ref[...] + a1_ref[...] + u_ref[...]) * dinv_ref[...]
    out_ref[...] = t[:, :C] + b_ref[...]


def _tc3(acc2, u, dinv, b_out):
    nb = NP // _BR
    return pl.pallas_call(
        _tc3_body,
        grid=(nb,),
        in_specs=[
            pl.BlockSpec((_BR, CP), lambda i: (i, 0)),
            pl.BlockSpec((_BR, CP), lambda i: (i + nb, 0)),
            pl.BlockSpec((_BR, CP), lambda i: (i, 0)),
            pl.BlockSpec((_BR, 1), lambda i: (i, 0)),
            pl.BlockSpec((1, C), lambda i: (0, 0)),
        ],
        out_specs=pl.BlockSpec((_BR, C), lambda i: (i, 0)),
        out_shape=jax.ShapeDtypeStruct((NP, C), jnp.float32),
    )(acc2, acc2, u, dinv, b_out)


def kernel(x_list, adj, W_hidden, b_hidden, W_out, b_out):
    src = jnp.concatenate([adj[0], jnp.zeros((EP - E,), jnp.int32)])
    dst = jnp.concatenate([adj[1], jnp.full((EP - E,), NP - 1, jnp.int32)])
    src2d = src.reshape(NCHUNK, CH)
    dst2d = dst.reshape(NCHUNK, CH)
    xp = jnp.zeros((K, NP, D), jnp.float32).at[:, :N].set(x_list)
    wo_pad = jnp.zeros((K * H, CP), jnp.float32).at[:, :C].set(W_out)
    zeros128 = jnp.zeros((NP, H), jnp.float32)
    zeros64 = jnp.zeros((NP, CP), jnp.float32)

    cnt32 = _count_kernel(dst)
    g0, g1, dinv = _tc1(cnt32, xp, W_hidden)

    acc = _prop128_kernel(g0, g1, src2d, dst2d, zeros128)
    u = _tc2(acc, g0, g1, dinv, b_hidden, wo_pad)

    acc2 = _prop64_kernel(u, src2d, dst2d, zeros64)
    out = _tc3(acc2, u, dinv, b_out.reshape(1, C))
    return out[:N]


# speedup vs baseline: 3.3992x; 1.4966x over previous
import functools

import jax
import jax.numpy as jnp
from jax import lax
from jax.experimental import pallas as pl
from jax.experimental.pallas import tpu as pltpu
from jax.experimental.pallas import tpu_sc as plsc

N = 10000
E = 320000
D = 128
H = 128
K = 2
C = 40

NP = 10240
CH = 128
NCHUNK = 2560
EP = NCHUNK * CH
CP = 48

_MESH = plsc.VectorSubcoreMesh(
    core_axis_name="c", subcore_axis_name="s", num_cores=2, num_subcores=16
)


def _count_body(dst_flat, cnt_out, idx_v, cnt_v):
    c = lax.axis_index("c")
    s = lax.axis_index("s")
    wid = s * 2 + c
    pltpu.sync_copy(dst_flat.at[pl.ds(wid * (EP // 32), EP // 32)], idx_v)

    def zero(i, carry):
        cnt_v[pl.ds(i * 16, 16)] = jnp.zeros((16,), jnp.float32)
        return carry

    lax.fori_loop(0, NP // 16, zero, 0)

    ones = jnp.ones((16,), jnp.float32)

    def body(i, carry):
        idx = idx_v[pl.ds(i * 16, 16)]
        plsc.addupdate_scatter(cnt_v, [idx], ones)
        return carry

    lax.fori_loop(0, EP // 32 // 16, body, 0)
    pltpu.sync_copy(cnt_v, cnt_out.at[wid])


def _count_kernel(dst_flat):
    return pl.kernel(
        _count_body,
        out_type=jax.ShapeDtypeStruct((32, NP), jnp.float32),
        mesh=_MESH,
        scratch_types=[
            pltpu.VMEM((EP // 32,), jnp.int32),
            pltpu.VMEM((NP,), jnp.float32),
        ],
        compiler_params=pltpu.CompilerParams(needs_layout_passes=False),
    )(dst_flat)


def _pipelined_block(table, acc_sp, src_v, dst_v, buf0, buf1, sem0, sem1, nch):
    pltpu.async_copy(table.at[src_v.at[0]], buf0, sem0)

    def inner(jj, carry):
        j0 = 2 * jj
        pltpu.async_copy(table.at[src_v.at[j0 + 1]], buf1, sem1)
        pltpu.make_async_copy(table.at[src_v.at[j0]], buf0, sem0).wait()
        pltpu.sync_copy(buf0, acc_sp.at[dst_v.at[j0]], add=True)

        @pl.when(jj + 1 < nch // 2)
        def _():
            pltpu.async_copy(table.at[src_v.at[j0 + 2]], buf0, sem0)

        pltpu.make_async_copy(table.at[src_v.at[j0 + 1]], buf1, sem1).wait()
        pltpu.sync_copy(buf1, acc_sp.at[dst_v.at[j0 + 1]], add=True)
        return carry

    lax.fori_loop(0, nch // 2, inner, 0)


def _prop128_body(g0, g1, src2d, dst2d, zeros_hbm, acc_out,
                  src_v, dst_v, buf0, buf1, sem0, sem1, acc_sp):
    c = lax.axis_index("c")
    s = lax.axis_index("s")
    per_tile = NCHUNK // 16
    rows = NP // 16
    pltpu.sync_copy(zeros_hbm.at[pl.ds(s * rows, rows)],
                    acc_sp.at[pl.ds(s * rows, rows)])
    plsc.subcore_barrier()

    def run(table):
        def outer(ob, carry):
            base = s * per_tile + ob * _SB
            pltpu.sync_copy(src2d.at[pl.ds(base, _SB)], src_v)
            pltpu.sync_copy(dst2d.at[pl.ds(base, _SB)], dst_v)
            _pipelined_block(table, acc_sp, src_v, dst_v,
                             buf0, buf1, sem0, sem1, _SB)
            return carry

        lax.fori_loop(0, per_tile // _SB, outer, 0)

    @pl.when(c == 0)
    def _():
        run(g0)

    @pl.when(c == 1)
    def _():
        run(g1)

    plsc.subcore_barrier()
    pltpu.sync_copy(acc_sp.at[pl.ds(s * rows, rows)],
                    acc_out.at[pl.ds(c * NP + s * rows, rows)])


_SB = 32


def _prop128_kernel(g0, g1, src2d, dst2d, zeros_hbm):
    return pl.kernel(
        _prop128_body,
        out_type=jax.ShapeDtypeStruct((2 * NP, H), jnp.bfloat16),
        mesh=_MESH,
        scratch_types=[
            pltpu.VMEM((_SB, CH), jnp.int32),
            pltpu.VMEM((_SB, CH), jnp.int32),
            pltpu.VMEM((CH, H), jnp.bfloat16),
            pltpu.VMEM((CH, H), jnp.bfloat16),
            pltpu.SemaphoreType.DMA,
            pltpu.SemaphoreType.DMA,
            pltpu.VMEM_SHARED((NP, H), jnp.bfloat16),
        ],
        compiler_params=pltpu.CompilerParams(use_tc_tiling_on_sc=False),
    )(g0, g1, src2d, dst2d, zeros_hbm)


def _prop64_body(u_hbm, src2d, dst2d, zeros_hbm, acc_out,
                 src_v, dst_v, buf0, buf1, sem0, sem1, acc_sp):
    c = lax.axis_index("c")
    s = lax.axis_index("s")
    per_tile = NCHUNK // 32
    tile_base = c * (NCHUNK // 2) + s * per_tile
    rows = NP // 16
    pltpu.sync_copy(zeros_hbm.at[pl.ds(s * rows, rows)],
                    acc_sp.at[pl.ds(s * rows, rows)])
    plsc.subcore_barrier()

    def outer(ob, carry):
        base = tile_base + ob * _SB2
        pltpu.sync_copy(src2d.at[pl.ds(base, _SB2)], src_v)
        pltpu.sync_copy(dst2d.at[pl.ds(base, _SB2)], dst_v)
        _pipelined_block(u_hbm, acc_sp, src_v, dst_v,
                         buf0, buf1, sem0, sem1, _SB2)
        return carry

    lax.fori_loop(0, per_tile // _SB2, outer, 0)

    plsc.subcore_barrier()
    pltpu.sync_copy(acc_sp.at[pl.ds(s * rows, rows)],
                    acc_out.at[pl.ds(c * NP + s * rows, rows)])


_SB2 = 16


def _prop64_kernel(u, src2d, dst2d, zeros64):
    return pl.kernel(
        _prop64_body,
        out_type=jax.ShapeDtypeStruct((2 * NP, CP), jnp.float32),
        mesh=_MESH,
        scratch_types=[
            pltpu.VMEM((_SB2, CH), jnp.int32),
            pltpu.VMEM((_SB2, CH), jnp.int32),
            pltpu.VMEM((CH, CP), jnp.float32),
            pltpu.VMEM((CH, CP), jnp.float32),
            pltpu.SemaphoreType.DMA,
            pltpu.SemaphoreType.DMA,
            pltpu.VMEM_SHARED((NP, CP), jnp.float32),
        ],
        compiler_params=pltpu.CompilerParams(use_tc_tiling_on_sc=False),
    )(u, src2d, dst2d, zeros64)


_BR = 512


def _tc1_body(cnt_ref, x_ref, w_ref, g0_ref, g1_ref, dinv_ref):
    cnt = jnp.sum(cnt_ref[...], axis=0)
    dinv = lax.rsqrt(cnt + 1.0)
    dcol = dinv[:, None]
    h0 = jnp.dot(x_ref[0], w_ref[0], preferred_element_type=jnp.float32)
    h1 = jnp.dot(x_ref[1], w_ref[1], preferred_element_type=jnp.float32)
    g0_ref[...] = h0 * dcol
    g1_ref[...] = h1 * dcol
    dinv_ref[...] = dcol


def _tc1(cnt32, xp, W_hidden):
    return pl.pallas_call(
        _tc1_body,
        grid=(NP // _BR,),
        in_specs=[
            pl.BlockSpec((32, _BR), lambda i: (0, i)),
            pl.BlockSpec((K, _BR, D), lambda i: (0, i, 0)),
            pl.BlockSpec((K, D, H), lambda i: (0, 0, 0)),
        ],
        out_specs=[
            pl.BlockSpec((_BR, H), lambda i: (i, 0)),
            pl.BlockSpec((_BR, H), lambda i: (i, 0)),
            pl.BlockSpec((_BR, 1), lambda i: (i, 0)),
        ],
        out_shape=[
            jax.ShapeDtypeStruct((NP, H), jnp.float32),
            jax.ShapeDtypeStruct((NP, H), jnp.float32),
            jax.ShapeDtypeStruct((NP, 1), jnp.float32),
        ],
    )(cnt32, xp, W_hidden)


def _tc2_body(a0_ref, a1_ref, g0_ref, g1_ref, dinv_ref, b_ref, wo_ref, u_ref):
    dinv = dinv_ref[...]
    a0 = a0_ref[...].astype(jnp.float32)
    a1 = a1_ref[...].astype(jnp.float32)
    h0 = jnp.maximum((a0 + g0_ref[...]) * dinv + b_ref[0], 0.0)
    h1 = jnp.maximum((a1 + g1_ref[...]) * dinv + b_ref[1], 0.0)
    wo = wo_ref[...]
    z = jnp.dot(h0, wo[:H], preferred_element_type=jnp.float32)
    z = z + jnp.dot(h1, wo[H:], preferred_element_type=jnp.float32)
    u_ref[...] = z * dinv


def _tc2(acc, g0, g1, dinv, b_hidden, wo_pad):
    nb = NP // _BR
    return pl.pallas_call(
        _tc2_body,
        grid=(nb,),
        in_specs=[
            pl.BlockSpec((_BR, H), lambda i: (i, 0)),
            pl.BlockSpec((_BR, H), lambda i: (i + nb, 0)),
            pl.BlockSpec((_BR, H), lambda i: (i, 0)),
            pl.BlockSpec((_BR, H), lambda i: (i, 0)),
            pl.BlockSpec((_BR, 1), lambda i: (i, 0)),
            pl.BlockSpec((K, H), lambda i: (0, 0)),
            pl.BlockSpec((K * H, CP), lambda i: (0, 0)),
        ],
        out_specs=pl.BlockSpec((_BR, CP), lambda i: (i, 0)),
        out_shape=jax.ShapeDtypeStruct((NP, CP), jnp.float32),
    )(acc, acc, g0, g1, dinv, b_hidden, wo_pad)


def _tc3_body(a0_ref, a1_ref, u_ref, dinv_ref, b_ref, out_ref):
    t = (a0_ref[...] + a1_ref[...] + u_ref[...]) * dinv_ref[...]
    out_ref[...] = t[:, :C] + b_ref[...]


def _tc3(acc2, u, dinv, b_out):
    nb = NP // _BR
    return pl.pallas_call(
        _tc3_body,
        grid=(nb,),
        in_specs=[
            pl.BlockSpec((_BR, CP), lambda i: (i, 0)),
            pl.BlockSpec((_BR, CP), lambda i: (i + nb, 0)),
            pl.BlockSpec((_BR, CP), lambda i: (i, 0)),
            pl.BlockSpec((_BR, 1), lambda i: (i, 0)),
            pl.BlockSpec((1, C), lambda i: (0, 0)),
        ],
        out_specs=pl.BlockSpec((_BR, C), lambda i: (i, 0)),
        out_shape=jax.ShapeDtypeStruct((NP, C), jnp.float32),
    )(acc2, acc2, u, dinv, b_out)


def kernel(x_list, adj, W_hidden, b_hidden, W_out, b_out):
    src = jnp.concatenate([adj[0], jnp.zeros((EP - E,), jnp.int32)])
    dst = jnp.concatenate([adj[1], jnp.full((EP - E,), NP - 1, jnp.int32)])
    src2d = src.reshape(NCHUNK, CH)
    dst2d = dst.reshape(NCHUNK, CH)
    xp = jnp.zeros((K, NP, D), jnp.float32).at[:, :N].set(x_list)
    wo_pad = jnp.zeros((K * H, CP), jnp.float32).at[:, :C].set(W_out)
    zeros128 = jnp.zeros((NP, H), jnp.bfloat16)
    zeros64 = jnp.zeros((NP, CP), jnp.float32)

    cnt32 = _count_kernel(dst)
    g0, g1, dinv = _tc1(cnt32, xp, W_hidden)

    acc = _prop128_kernel(g0.astype(jnp.bfloat16), g1.astype(jnp.bfloat16),
                          src2d, dst2d, zeros128)
    u = _tc2(acc, g0, g1, dinv, b_hidden, wo_pad)

    acc2 = _prop64_kernel(u, src2d, dst2d, zeros64)
    out = _tc3(acc2, u, dinv, b_out.reshape(1, C))
    return out[:N]


# speedup vs baseline: 3.6826x; 1.0834x over previous
import functools

import jax
import jax.numpy as jnp
from jax import lax
from jax.experimental import pallas as pl
from jax.experimental.pallas import tpu as pltpu
from jax.experimental.pallas import tpu_sc as plsc

N = 10000
E = 320000
D = 128
H = 128
K = 2
C = 40

NP = 10240
CH = 128
NCHUNK = 2560
EP = NCHUNK * CH
CP = 64

_MESH = plsc.VectorSubcoreMesh(
    core_axis_name="c", subcore_axis_name="s", num_cores=2, num_subcores=16
)


def _count_body(dst_flat, cnt_out, idx_v, cnt_v):
    c = lax.axis_index("c")
    s = lax.axis_index("s")
    wid = s * 2 + c
    pltpu.sync_copy(dst_flat.at[pl.ds(wid * (EP // 32), EP // 32)], idx_v)

    def zero(i, carry):
        cnt_v[pl.ds(i * 16, 16)] = jnp.zeros((16,), jnp.float32)
        return carry

    lax.fori_loop(0, NP // 16, zero, 0)

    ones = jnp.ones((16,), jnp.float32)

    def body(i, carry):
        idx = idx_v[pl.ds(i * 16, 16)]
        plsc.addupdate_scatter(cnt_v, [idx], ones)
        return carry

    lax.fori_loop(0, EP // 32 // 16, body, 0)
    pltpu.sync_copy(cnt_v, cnt_out.at[wid])


def _count_kernel(dst_flat):
    return pl.kernel(
        _count_body,
        out_type=jax.ShapeDtypeStruct((32, NP), jnp.float32),
        mesh=_MESH,
        scratch_types=[
            pltpu.VMEM((EP // 32,), jnp.int32),
            pltpu.VMEM((NP,), jnp.float32),
        ],
        compiler_params=pltpu.CompilerParams(needs_layout_passes=False),
    )(dst_flat)


def _pipelined_block(table, acc_sp, src_v, dst_v, buf0, buf1, sem0, sem1, nch):
    pltpu.async_copy(table.at[src_v.at[0]], buf0, sem0)

    def inner(jj, carry):
        j0 = 2 * jj
        pltpu.async_copy(table.at[src_v.at[j0 + 1]], buf1, sem1)
        pltpu.make_async_copy(table.at[src_v.at[j0]], buf0, sem0).wait()
        pltpu.sync_copy(buf0, acc_sp.at[dst_v.at[j0]], add=True)

        @pl.when(jj + 1 < nch // 2)
        def _():
            pltpu.async_copy(table.at[src_v.at[j0 + 2]], buf0, sem0)

        pltpu.make_async_copy(table.at[src_v.at[j0 + 1]], buf1, sem1).wait()
        pltpu.sync_copy(buf1, acc_sp.at[dst_v.at[j0 + 1]], add=True)
        return carry

    lax.fori_loop(0, nch // 2, inner, 0)


def _prop128_body(g0, g1, src2d, dst2d, zeros_hbm, acc_out,
                  src_v, dst_v, buf0, buf1, sem0, sem1, acc_sp):
    c = lax.axis_index("c")
    s = lax.axis_index("s")
    per_tile = NCHUNK // 16
    rows = NP // 16
    pltpu.sync_copy(zeros_hbm.at[pl.ds(s * rows, rows)],
                    acc_sp.at[pl.ds(s * rows, rows)])
    plsc.subcore_barrier()

    def run(table):
        def outer(ob, carry):
            base = s * per_tile + ob * _SB
            pltpu.sync_copy(src2d.at[pl.ds(base, _SB)], src_v)
            pltpu.sync_copy(dst2d.at[pl.ds(base, _SB)], dst_v)
            _pipelined_block(table, acc_sp, src_v, dst_v,
                             buf0, buf1, sem0, sem1, _SB)
            return carry

        lax.fori_loop(0, per_tile // _SB, outer, 0)

    @pl.when(c == 0)
    def _():
        run(g0)

    @pl.when(c == 1)
    def _():
        run(g1)

    plsc.subcore_barrier()
    pltpu.sync_copy(acc_sp.at[pl.ds(s * rows, rows)],
                    acc_out.at[pl.ds(c * NP + s * rows, rows)])


_SB = 32


def _prop128_kernel(g0, g1, src2d, dst2d, zeros_hbm):
    return pl.kernel(
        _prop128_body,
        out_type=jax.ShapeDtypeStruct((2 * NP, H), jnp.bfloat16),
        mesh=_MESH,
        scratch_types=[
            pltpu.VMEM((_SB, CH), jnp.int32),
            pltpu.VMEM((_SB, CH), jnp.int32),
            pltpu.VMEM((CH, H), jnp.bfloat16),
            pltpu.VMEM((CH, H), jnp.bfloat16),
            pltpu.SemaphoreType.DMA,
            pltpu.SemaphoreType.DMA,
            pltpu.VMEM_SHARED((NP, H), jnp.bfloat16),
        ],
        compiler_params=pltpu.CompilerParams(use_tc_tiling_on_sc=False),
    )(g0, g1, src2d, dst2d, zeros_hbm)


def _prop64_body(u_hbm, src2d, dst2d, zeros_hbm, acc_out,
                 src_v, dst_v, buf0, buf1, sem0, sem1, acc_sp):
    c = lax.axis_index("c")
    s = lax.axis_index("s")
    per_tile = NCHUNK // 32
    tile_base = c * (NCHUNK // 2) + s * per_tile
    rows = NP // 16
    pltpu.sync_copy(zeros_hbm.at[pl.ds(s * rows, rows)],
                    acc_sp.at[pl.ds(s * rows, rows)])
    plsc.subcore_barrier()

    def outer(ob, carry):
        base = tile_base + ob * _SB2
        pltpu.sync_copy(src2d.at[pl.ds(base, _SB2)], src_v)
        pltpu.sync_copy(dst2d.at[pl.ds(base, _SB2)], dst_v)
        _pipelined_block(u_hbm, acc_sp, src_v, dst_v,
                         buf0, buf1, sem0, sem1, _SB2)
        return carry

    lax.fori_loop(0, per_tile // _SB2, outer, 0)

    plsc.subcore_barrier()
    pltpu.sync_copy(acc_sp.at[pl.ds(s * rows, rows)],
                    acc_out.at[pl.ds(c * NP + s * rows, rows)])


_SB2 = 16


def _prop64_kernel(u, src2d, dst2d, zeros64):
    return pl.kernel(
        _prop64_body,
        out_type=jax.ShapeDtypeStruct((2 * NP, CP), jnp.bfloat16),
        mesh=_MESH,
        scratch_types=[
            pltpu.VMEM((_SB2, CH), jnp.int32),
            pltpu.VMEM((_SB2, CH), jnp.int32),
            pltpu.VMEM((CH, CP), jnp.bfloat16),
            pltpu.VMEM((CH, CP), jnp.bfloat16),
            pltpu.SemaphoreType.DMA,
            pltpu.SemaphoreType.DMA,
            pltpu.VMEM_SHARED((NP, CP), jnp.bfloat16),
        ],
        compiler_params=pltpu.CompilerParams(use_tc_tiling_on_sc=False),
    )(u, src2d, dst2d, zeros64)


_BR = 512


def _tc1_body(cnt_ref, x_ref, w_ref, g0_ref, g1_ref, dinv_ref):
    cnt = jnp.sum(cnt_ref[...], axis=0)
    dinv = lax.rsqrt(cnt + 1.0)
    dcol = dinv[:, None]
    h0 = jnp.dot(x_ref[0], w_ref[0], preferred_element_type=jnp.float32)
    h1 = jnp.dot(x_ref[1], w_ref[1], preferred_element_type=jnp.float32)
    g0_ref[...] = h0 * dcol
    g1_ref[...] = h1 * dcol
    dinv_ref[...] = dcol


def _tc1(cnt32, xp, W_hidden):
    return pl.pallas_call(
        _tc1_body,
        grid=(NP // _BR,),
        in_specs=[
            pl.BlockSpec((32, _BR), lambda i: (0, i)),
            pl.BlockSpec((K, _BR, D), lambda i: (0, i, 0)),
            pl.BlockSpec((K, D, H), lambda i: (0, 0, 0)),
        ],
        out_specs=[
            pl.BlockSpec((_BR, H), lambda i: (i, 0)),
            pl.BlockSpec((_BR, H), lambda i: (i, 0)),
            pl.BlockSpec((_BR, 1), lambda i: (i, 0)),
        ],
        out_shape=[
            jax.ShapeDtypeStruct((NP, H), jnp.float32),
            jax.ShapeDtypeStruct((NP, H), jnp.float32),
            jax.ShapeDtypeStruct((NP, 1), jnp.float32),
        ],
    )(cnt32, xp, W_hidden)


def _tc2_body(a0_ref, a1_ref, g0_ref, g1_ref, dinv_ref, b_ref, wo_ref, u_ref):
    dinv = dinv_ref[...]
    a0 = a0_ref[...].astype(jnp.float32)
    a1 = a1_ref[...].astype(jnp.float32)
    h0 = jnp.maximum((a0 + g0_ref[...]) * dinv + b_ref[0], 0.0)
    h1 = jnp.maximum((a1 + g1_ref[...]) * dinv + b_ref[1], 0.0)
    wo = wo_ref[...]
    z = jnp.dot(h0, wo[:H], preferred_element_type=jnp.float32)
    z = z + jnp.dot(h1, wo[H:], preferred_element_type=jnp.float32)
    u_ref[...] = z * dinv


def _tc2(acc, g0, g1, dinv, b_hidden, wo_pad):
    nb = NP // _BR
    return pl.pallas_call(
        _tc2_body,
        grid=(nb,),
        in_specs=[
            pl.BlockSpec((_BR, H), lambda i: (i, 0)),
            pl.BlockSpec((_BR, H), lambda i: (i + nb, 0)),
            pl.BlockSpec((_BR, H), lambda i: (i, 0)),
            pl.BlockSpec((_BR, H), lambda i: (i, 0)),
            pl.BlockSpec((_BR, 1), lambda i: (i, 0)),
            pl.BlockSpec((K, H), lambda i: (0, 0)),
            pl.BlockSpec((K * H, CP), lambda i: (0, 0)),
        ],
        out_specs=pl.BlockSpec((_BR, CP), lambda i: (i, 0)),
        out_shape=jax.ShapeDtypeStruct((NP, CP), jnp.float32),
    )(acc, acc, g0, g1, dinv, b_hidden, wo_pad)


def _tc3_body(a0_ref, a1_ref, u_ref, dinv_ref, b_ref, out_ref):
    a2 = a0_ref[...].astype(jnp.float32) + a1_ref[...].astype(jnp.float32)
    t = (a2 + u_ref[...]) * dinv_ref[...]
    out_ref[...] = t[:, :C] + b_ref[...]


def _tc3(acc2, u, dinv, b_out):
    nb = NP // _BR
    return pl.pallas_call(
        _tc3_body,
        grid=(nb,),
        in_specs=[
            pl.BlockSpec((_BR, CP), lambda i: (i, 0)),
            pl.BlockSpec((_BR, CP), lambda i: (i + nb, 0)),
            pl.BlockSpec((_BR, CP), lambda i: (i, 0)),
            pl.BlockSpec((_BR, 1), lambda i: (i, 0)),
            pl.BlockSpec((1, C), lambda i: (0, 0)),
        ],
        out_specs=pl.BlockSpec((_BR, C), lambda i: (i, 0)),
        out_shape=jax.ShapeDtypeStruct((NP, C), jnp.float32),
    )(acc2, acc2, u, dinv, b_out)


def kernel(x_list, adj, W_hidden, b_hidden, W_out, b_out):
    src = jnp.concatenate([adj[0], jnp.zeros((EP - E,), jnp.int32)])
    dst = jnp.concatenate([adj[1], jnp.full((EP - E,), NP - 1, jnp.int32)])
    src2d = src.reshape(NCHUNK, CH)
    dst2d = dst.reshape(NCHUNK, CH)
    xp = jnp.zeros((K, NP, D), jnp.float32).at[:, :N].set(x_list)
    wo_pad = jnp.zeros((K * H, CP), jnp.float32).at[:, :C].set(W_out)
    zeros128 = jnp.zeros((NP, H), jnp.bfloat16)
    zeros64 = jnp.zeros((NP, CP), jnp.bfloat16)

    cnt32 = _count_kernel(dst)
    g0, g1, dinv = _tc1(cnt32, xp, W_hidden)

    acc = _prop128_kernel(g0.astype(jnp.bfloat16), g1.astype(jnp.bfloat16),
                          src2d, dst2d, zeros128)
    u = _tc2(acc, g0, g1, dinv, b_hidden, wo_pad)

    acc2 = _prop64_kernel(u.astype(jnp.bfloat16), src2d, dst2d, zeros64)
    out = _tc3(acc2, u, dinv, b_out.reshape(1, C))
    return out[:N]
